# baseline (speedup 1.0000x reference)
import functools

import jax
import jax.numpy as jnp
from jax import lax
from jax.experimental import pallas as pl
from jax.experimental.pallas import tpu as pltpu
from jax.experimental.pallas import tpu_sc as plsc

_DIM = 64
_LANES = 16
_NCORES = 2
_NSUB = 16
_NW = _NCORES * _NSUB
_GCHUNK = 128


def _make_sc_call(batch):
    bpw = batch // _NW
    nch = bpw // _GCHUNK
    mesh = plsc.VectorSubcoreMesh(core_axis_name="c", subcore_axis_name="s")

    @functools.partial(
        pl.kernel,
        mesh=mesh,
        out_type=jax.ShapeDtypeStruct((_NW, _LANES), jnp.float32),
        compiler_params=pltpu.CompilerParams(use_tc_tiling_on_sc=False),
        scratch_types=[
            pltpu.VMEM((nch, _GCHUNK), jnp.int32),
            pltpu.VMEM((bpw, _DIM), jnp.float32),
            pltpu.VMEM((bpw, _DIM), jnp.float32),
            pltpu.VMEM((_LANES,), jnp.float32),
            pltpu.SemaphoreType.DMA,
            pltpu.SemaphoreType.DMA,
        ],
    )
    def sc_kernel(x_hbm, y_hbm, centers_hbm, out_hbm, idx_v, x_v, c_v, acc_v,
                  sem_x, sem_g):
        wid = lax.axis_index("s") * _NCORES + lax.axis_index("c")
        base = wid * bpw

        pltpu.sync_copy(y_hbm.at[pl.ds(wid * nch, nch)], idx_v)

        cp_x = pltpu.async_copy(x_hbm.at[pl.ds(base, bpw)], x_v, sem_x)
        gathers = []
        for j in range(nch):
            gathers.append(
                pltpu.async_copy(
                    centers_hbm.at[idx_v.at[j]],
                    c_v.at[pl.ds(j * _GCHUNK, _GCHUNK)],
                    sem_g,
                )
            )
        cp_x.wait()
        for g in gathers:
            g.wait()

        zeros = jnp.zeros((_LANES,), jnp.float32)

        @plsc.parallel_loop(0, bpw, 1, unroll=8,
                            carry=(zeros, zeros, zeros, zeros))
        def accs(r, accs_in):
            new = []
            for k in range(_DIM // _LANES):
                d = x_v[r, pl.ds(k * _LANES, _LANES)] - c_v[r, pl.ds(k * _LANES, _LANES)]
                new.append(accs_in[k] + d * d)
            return tuple(new)

        acc_v[...] = accs[0] + accs[1] + accs[2] + accs[3]
        pltpu.sync_copy(acc_v, out_hbm.at[wid])

    return sc_kernel


def kernel(x, y, centers):
    batch, dim = x.shape
    y2 = y.reshape(batch // _GCHUNK, _GCHUNK).astype(jnp.int32)
    partials = _make_sc_call(batch)(x, y2, centers)
    return jnp.sum(partials) / (batch * dim)


# speedup vs baseline: 2.3710x; 2.3710x over previous
import functools

import jax
import jax.numpy as jnp
from jax import lax
from jax.experimental import pallas as pl
from jax.experimental.pallas import tpu as pltpu
from jax.experimental.pallas import tpu_sc as plsc

_DIM = 64
_LANES = 16
_NCORES = 2
_NSUB = 16
_NW = _NCORES * _NSUB


def _make_sc_call(batch):
    bpw = batch // _NW
    nblk = bpw // _LANES
    mesh = plsc.VectorSubcoreMesh(core_axis_name="c", subcore_axis_name="s")

    @functools.partial(
        pl.kernel,
        mesh=mesh,
        out_type=jax.ShapeDtypeStruct((_NW, _LANES), jnp.float32),
        scratch_types=[
            pltpu.VMEM((bpw,), jnp.int32),
            pltpu.VMEM((bpw, _DIM), jnp.float32),
            pltpu.VMEM((_LANES, _DIM), jnp.float32),
            pltpu.VMEM((_LANES,), jnp.float32),
            pltpu.SemaphoreType.DMA,
            pltpu.SemaphoreType.DMA,
        ],
    )
    def sc_kernel(x_hbm, y_hbm, centers_hbm, out_hbm, idx_v, x_v,
                  c_v, acc_v, sem_x, sem_g):
        wid = lax.axis_index("s") * _NCORES + lax.axis_index("c")
        base = wid * bpw

        pltpu.sync_copy(y_hbm.at[pl.ds(base, bpw)], idx_v)
        cp_x = pltpu.async_copy(x_hbm.at[pl.ds(base, bpw)], x_v, sem_x)
        cp_x.wait()

        zeros = jnp.zeros((_LANES,), jnp.float32)

        def body(g, accs_in):
            off = g * _LANES
            rv = idx_v[pl.ds(off, _LANES)]
            tv = rv >> 3
            sv = rv & 7
            cps = []
            for i in range(_LANES):
                cps.append(pltpu.async_copy(
                    centers_hbm.at[tv[i], sv[i]],
                    c_v.at[i],
                    sem_g,
                ))
            for cp in cps:
                cp.wait()
            new = list(accs_in)
            for i in range(_LANES):
                for k in range(_DIM // _LANES):
                    d = (x_v[off + i, pl.ds(k * _LANES, _LANES)]
                         - c_v[i, pl.ds(k * _LANES, _LANES)])
                    new[k] = new[k] + d * d
            return tuple(new)

        accs = lax.fori_loop(0, nblk, body, (zeros, zeros, zeros, zeros))

        acc_v[...] = accs[0] + accs[1] + accs[2] + accs[3]
        pltpu.sync_copy(acc_v, out_hbm.at[wid])

    return sc_kernel


def kernel(x, y, centers):
    batch, dim = x.shape
    nrows = centers.shape[0]
    centers3 = centers.reshape(nrows // 8, 8, dim)
    partials = _make_sc_call(batch)(x, y.astype(jnp.int32), centers3)
    return jnp.sum(partials) / (batch * dim)


# speedup vs baseline: 2.4535x; 1.0348x over previous
import functools

import jax
import jax.numpy as jnp
from jax import lax
from jax.experimental import pallas as pl
from jax.experimental.pallas import tpu as pltpu
from jax.experimental.pallas import tpu_sc as plsc

_DIM = 64
_LANES = 16
_NCORES = 2
_NSUB = 16
_NW = _NCORES * _NSUB


def _make_sc_call(batch):
    bpw = batch // _NW
    nblk = bpw // _LANES
    mesh = plsc.VectorSubcoreMesh(core_axis_name="c", subcore_axis_name="s")

    @functools.partial(
        pl.kernel,
        mesh=mesh,
        out_type=jax.ShapeDtypeStruct((_NW, _LANES), jnp.float32),
        scratch_types=[
            pltpu.VMEM((bpw,), jnp.int32),
            pltpu.VMEM((bpw, _DIM), jnp.float32),
            pltpu.VMEM((_LANES, _DIM), jnp.float32),
            pltpu.VMEM((_LANES, _DIM), jnp.float32),
            pltpu.VMEM((_LANES,), jnp.float32),
            pltpu.SemaphoreType.DMA,
            pltpu.SemaphoreType.DMA,
            pltpu.SemaphoreType.DMA,
        ],
    )
    def sc_kernel(x_hbm, y_hbm, centers_hbm, out_hbm, idx_v, x_v,
                  c_v0, c_v1, acc_v, sem_x, sem_g0, sem_g1):
        wid = lax.axis_index("s") * _NCORES + lax.axis_index("c")
        base = wid * bpw

        pltpu.sync_copy(y_hbm.at[pl.ds(base, bpw)], idx_v)
        cp_x = pltpu.async_copy(x_hbm.at[pl.ds(base, bpw)], x_v, sem_x)

        def issue_block(g, cref, sem):
            rv = idx_v[pl.ds(g * _LANES, _LANES)]
            tv = rv >> 3
            sv = rv & 7
            for i in range(_LANES):
                pltpu.async_copy(
                    centers_hbm.at[tv[i], sv[i]], cref.at[i], sem)

        def drain_block(cref, sem):
            for i in range(_LANES):
                pltpu.make_async_copy(
                    centers_hbm.at[0, 0], cref.at[i], sem).wait()

        def compute_block(g, cref, accs_in):
            off = g * _LANES
            new = list(accs_in)
            for i in range(_LANES):
                for k in range(_DIM // _LANES):
                    d = (x_v[off + i, pl.ds(k * _LANES, _LANES)]
                         - cref[i, pl.ds(k * _LANES, _LANES)])
                    new[k] = new[k] + d * d
            return tuple(new)

        issue_block(0, c_v0, sem_g0)
        issue_block(1, c_v1, sem_g1)
        cp_x.wait()

        zeros = jnp.zeros((_LANES,), jnp.float32)

        def body(it, accs_in):
            g = it * 2
            accs = accs_in
            for b, (cref, sem) in enumerate(((c_v0, sem_g0), (c_v1, sem_g1))):
                drain_block(cref, sem)
                accs = compute_block(g + b, cref, accs)

                @pl.when(g + b + 2 < nblk)
                def _():
                    issue_block(g + b + 2, cref, sem)

            return accs

        accs = lax.fori_loop(0, nblk // 2, body, (zeros, zeros, zeros, zeros))

        acc_v[...] = accs[0] + accs[1] + accs[2] + accs[3]
        pltpu.sync_copy(acc_v, out_hbm.at[wid])

    return sc_kernel


def kernel(x, y, centers):
    batch, dim = x.shape
    nrows = centers.shape[0]
    centers3 = centers.reshape(nrows // 8, 8, dim)
    partials = _make_sc_call(batch)(x, y.astype(jnp.int32), centers3)
    return jnp.sum(partials) / (batch * dim)
